# Initial kernel scaffold; baseline (speedup 1.0000x reference)
import jax
import jax.numpy as jnp
from jax.experimental import pallas as pl


def kernel(x, edge_index, W1, b1, W2, b2):
    raise NotImplementedError("write your pallas kernel here")


# speedup vs baseline: 17.1479x; 17.1479x over previous
import jax
import jax.numpy as jnp
from jax import lax
from jax.experimental import pallas as pl
from jax.experimental.pallas import tpu as pltpu
from jax.experimental.pallas import tpu_sc as plsc

N = 10000
D_IN = 256
D_HID = 128
E = 160000

NC = 2
NS = 16
NW = NC * NS
CHUNK = 128
NPAD = 10240
RPT = NPAD // NS
DEAD = 10100
E_TOT = E + N
NCH = -(-E_TOT // (NW * CHUNK))
E_PAD = NW * NCH * CHUNK
BLK = 1024


def _sc_mesh():
    return plsc.VectorSubcoreMesh(
        core_axis_name="c", subcore_axis_name="s",
        num_cores=NC, num_subcores=NS)


_SC_PARAMS = pltpu.CompilerParams(needs_layout_passes=False)


def _zero_1d(ref, n):
    z16 = jnp.zeros((16,), jnp.float32)

    def body(i, carry):
        ref[pl.ds(i * 16, 16)] = z16
        return carry
    lax.fori_loop(0, n // 16, body, 0)


def _reduce_tiles(stage_s, sum_v, tmp_v, out_hbm, c, s):
    pltpu.sync_copy(stage_s.at[0, pl.ds(s * RPT, RPT)], sum_v)

    def jbody(j, carry):
        pltpu.sync_copy(stage_s.at[j, pl.ds(s * RPT, RPT)], tmp_v)

        def ibody(i, carry2):
            sl = pl.ds(i * 16, 16)
            sum_v[sl] = sum_v[sl] + tmp_v[sl]
            return carry2
        lax.fori_loop(0, RPT // 16, ibody, 0)
        return carry
    lax.fori_loop(1, NS, jbody, 0)
    pltpu.sync_copy(sum_v, out_hbm.at[c, pl.ds(s * RPT, RPT)])


def _deg_body(didx_hbm, out_hbm, didx_v, acc_v, tmp_v, sum_v, stage_s):
    c = lax.axis_index("c")
    s = lax.axis_index("s")
    wid = c * NS + s
    pltpu.sync_copy(didx_hbm.at[wid], didx_v)
    _zero_1d(acc_v, NPAD)
    ones16 = jnp.full((16,), 1.0, jnp.float32)

    def body(j, carry):
        for k in range(CHUNK // 16):
            idx = didx_v[j, pl.ds(k * 16, 16)]
            plsc.addupdate_scatter(acc_v, [idx], ones16)
        return carry
    lax.fori_loop(0, NCH, body, 0)
    pltpu.sync_copy(acc_v, stage_s.at[s])
    plsc.subcore_barrier()
    _reduce_tiles(stage_s, sum_v, tmp_v, out_hbm, c, s)


def _sc_degree(didx):
    return pl.kernel(
        _deg_body,
        out_type=jax.ShapeDtypeStruct((NC, NPAD), jnp.float32),
        mesh=_sc_mesh(),
        compiler_params=_SC_PARAMS,
        scratch_types=[
            pltpu.VMEM((NCH, CHUNK), jnp.int32),
            pltpu.VMEM((NPAD,), jnp.float32),
            pltpu.VMEM((RPT,), jnp.float32),
            pltpu.VMEM((RPT,), jnp.float32),
            pltpu.VMEM_SHARED((NS, NPAD), jnp.float32),
        ],
    )(didx)


def _spmm_body(y_hbm, sidx_hbm, didx_hbm, zrows_hbm, out_hbm,
               sidx_v, didx_v, rows_a, rows_b, acc_s, sem_a, sem_b):
    c = lax.axis_index("c")
    s = lax.axis_index("s")
    wid = c * NS + s
    pltpu.sync_copy(zrows_hbm, rows_a)
    for k in range(RPT // CHUNK):
        pltpu.sync_copy(rows_a, acc_s.at[pl.ds(s * RPT + k * CHUNK, CHUNK)])
    pltpu.sync_copy(sidx_hbm.at[wid], sidx_v)
    pltpu.sync_copy(didx_hbm.at[wid], didx_v)
    plsc.subcore_barrier()

    def body(i, carry):
        j = 2 * i
        cp_a = pltpu.async_copy(y_hbm.at[sidx_v.at[j]], rows_a, sem_a)
        cp_b = pltpu.async_copy(y_hbm.at[sidx_v.at[j + 1]], rows_b, sem_b)
        cp_a.wait()
        pltpu.sync_copy(rows_a, acc_s.at[didx_v.at[j]], add=True)
        cp_b.wait()
        pltpu.sync_copy(rows_b, acc_s.at[didx_v.at[j + 1]], add=True)
        return carry
    lax.fori_loop(0, NCH // 2, body, 0)
    plsc.subcore_barrier()
    for k in range(RPT // CHUNK):
        off = s * RPT + k * CHUNK
        pltpu.sync_copy(acc_s.at[pl.ds(off, CHUNK)], rows_a)
        pltpu.sync_copy(rows_a, out_hbm.at[c, pl.ds(off, CHUNK)])


def _sc_spmm(y, sidx, didx, zrows):
    return pl.kernel(
        _spmm_body,
        out_type=jax.ShapeDtypeStruct((NC, NPAD, D_HID), jnp.float32),
        mesh=_sc_mesh(),
        compiler_params=_SC_PARAMS,
        scratch_types=[
            pltpu.VMEM((NCH, CHUNK), jnp.int32),
            pltpu.VMEM((NCH, CHUNK), jnp.int32),
            pltpu.VMEM((CHUNK, D_HID), jnp.float32),
            pltpu.VMEM((CHUNK, D_HID), jnp.float32),
            pltpu.VMEM_SHARED((NPAD, D_HID), jnp.float32),
            pltpu.SemaphoreType.DMA,
            pltpu.SemaphoreType.DMA,
        ],
    )(y, sidx, didx, zrows)


def _seg_body(z_hbm, sidx_hbm, didx_hbm, out_hbm,
              sidx_v, didx_v, z_v, acc_v, tmp_v, sum_v, stage_s):
    c = lax.axis_index("c")
    s = lax.axis_index("s")
    wid = c * NS + s
    pltpu.sync_copy(z_hbm, z_v)
    pltpu.sync_copy(sidx_hbm.at[wid], sidx_v)
    pltpu.sync_copy(didx_hbm.at[wid], didx_v)
    _zero_1d(acc_v, NPAD)

    def body(j, carry):
        for k in range(CHUNK // 16):
            sl = pl.ds(k * 16, 16)
            svals = plsc.load_gather(z_v, [sidx_v[j, sl]])
            plsc.addupdate_scatter(acc_v, [didx_v[j, sl]], svals)
        return carry
    lax.fori_loop(0, NCH, body, 0)
    pltpu.sync_copy(acc_v, stage_s.at[s])
    plsc.subcore_barrier()
    _reduce_tiles(stage_s, sum_v, tmp_v, out_hbm, c, s)


def _sc_seg(z, sidx, didx):
    return pl.kernel(
        _seg_body,
        out_type=jax.ShapeDtypeStruct((NC, NPAD), jnp.float32),
        mesh=_sc_mesh(),
        compiler_params=_SC_PARAMS,
        scratch_types=[
            pltpu.VMEM((NCH, CHUNK), jnp.int32),
            pltpu.VMEM((NCH, CHUNK), jnp.int32),
            pltpu.VMEM((NPAD,), jnp.float32),
            pltpu.VMEM((NPAD,), jnp.float32),
            pltpu.VMEM((RPT,), jnp.float32),
            pltpu.VMEM((RPT,), jnp.float32),
            pltpu.VMEM_SHARED((NS, NPAD), jnp.float32),
        ],
    )(z, sidx, didx)


def _front_body(x_ref, w1_ref, degp_ref, y_ref, dinv_ref):
    b = pl.program_id(0)
    deg = degp_ref[0] + degp_ref[1]
    rows = b * BLK + lax.broadcasted_iota(jnp.int32, (BLK, 1), 0)
    dinv = jnp.where(rows < N, lax.rsqrt(jnp.maximum(deg, 1.0)), 0.0)
    xw = jnp.dot(x_ref[...], w1_ref[...], preferred_element_type=jnp.float32)
    y_ref[...] = xw * dinv
    dinv_ref[...] = dinv


def _tc_front(x_pad, w1, deg_parts):
    return pl.pallas_call(
        _front_body,
        grid=(NPAD // BLK,),
        in_specs=[
            pl.BlockSpec((BLK, D_IN), lambda b: (b, 0)),
            pl.BlockSpec((D_IN, D_HID), lambda b: (0, 0)),
            pl.BlockSpec((NC, BLK, 1), lambda b: (0, b, 0)),
        ],
        out_specs=[
            pl.BlockSpec((BLK, D_HID), lambda b: (b, 0)),
            pl.BlockSpec((BLK, 1), lambda b: (b, 0)),
        ],
        out_shape=[
            jax.ShapeDtypeStruct((NPAD, D_HID), jnp.float32),
            jax.ShapeDtypeStruct((NPAD, 1), jnp.float32),
        ],
    )(x_pad, w1, deg_parts)


def _mid_body(s1p_ref, dinv_ref, b1_ref, w2_ref, z_ref):
    h = (s1p_ref[0] + s1p_ref[1]) * dinv_ref[...] + b1_ref[...]
    t = jnp.tanh(h)
    hw = jnp.dot(t, w2_ref[...], preferred_element_type=jnp.float32)
    z_ref[...] = hw * dinv_ref[...]


def _tc_mid(s1_parts, dinv, b1_row, w2):
    return pl.pallas_call(
        _mid_body,
        grid=(NPAD // BLK,),
        in_specs=[
            pl.BlockSpec((NC, BLK, D_HID), lambda b: (0, b, 0)),
            pl.BlockSpec((BLK, 1), lambda b: (b, 0)),
            pl.BlockSpec((1, D_HID), lambda b: (0, 0)),
            pl.BlockSpec((D_HID, 1), lambda b: (0, 0)),
        ],
        out_specs=pl.BlockSpec((BLK, 1), lambda b: (b, 0)),
        out_shape=jax.ShapeDtypeStruct((NPAD, 1), jnp.float32),
    )(s1_parts, dinv, b1_row, w2)


def _tail_body(s2p_ref, dinv_ref, b2_ref, out_ref):
    h2 = (s2p_ref[0] + s2p_ref[1]) * dinv_ref[...] + b2_ref[...]
    rows = lax.broadcasted_iota(jnp.int32, (NPAD, 1), 0)
    sq = jnp.where(rows < N, h2 * h2, 0.0)
    out_ref[...] = jnp.sqrt(jnp.sum(sq, axis=(0, 1), keepdims=True))


def _tc_tail(s2_parts, dinv, b2_2d):
    return pl.pallas_call(
        _tail_body,
        out_shape=jax.ShapeDtypeStruct((1, 1), jnp.float32),
    )(s2_parts, dinv, b2_2d)


def kernel(x, edge_index, W1, b1, W2, b2):
    src = edge_index[0].astype(jnp.int32)
    dst = edge_index[1].astype(jnp.int32)
    loop = jnp.arange(N, dtype=jnp.int32)
    pad = jnp.full((E_PAD - E_TOT,), DEAD, dtype=jnp.int32)
    sidx = jnp.concatenate([src, loop, pad]).reshape(NW, NCH, CHUNK)
    didx = jnp.concatenate([dst, loop, pad]).reshape(NW, NCH, CHUNK)

    x_pad = jnp.pad(x, ((0, NPAD - N), (0, 0)))
    zrows = jnp.zeros((CHUNK, D_HID), jnp.float32)
    b1_row = b1.reshape(1, D_HID)
    b2_2d = b2.reshape(1, 1)

    deg_parts = _sc_degree(didx).reshape(NC, NPAD, 1)
    y, dinv = _tc_front(x_pad, W1, deg_parts)
    s1_parts = _sc_spmm(y, sidx, didx, zrows)
    z = _tc_mid(s1_parts, dinv, b1_row, W2)
    s2_parts = _sc_seg(z.reshape(NPAD), sidx, didx).reshape(NC, NPAD, 1)
    out = _tc_tail(s2_parts, dinv, b2_2d)
    return out.reshape(1)
